# baseline (speedup 1.0000x reference)
import functools

import jax
import jax.numpy as jnp
from jax import lax
from jax.experimental import pallas as pl
from jax.experimental.pallas import tpu as pltpu
from jax.experimental.pallas import tpu_sc as plsc

D = 64
LANES = 16
CHUNK = 100
NBUF = 4


@functools.lru_cache(maxsize=None)
def _build(n_chunks: int, vocab: int, max_seq: int):
    mesh = plsc.VectorSubcoreMesh(core_axis_name="c", subcore_axis_name="s")
    info = plsc.get_sparse_core_info()
    nc, ns = info.num_cores, info.num_subcores
    nw = nc * ns
    assert n_chunks % (nw * NBUF) == 0
    chunks_per_w = n_chunks // nw
    n_blks = chunks_per_w // NBUF
    n_seq = n_chunks // 2
    seq_per_w = n_seq // nw

    @functools.partial(
        pl.kernel,
        out_type=jax.ShapeDtypeStruct((n_seq, 2 * CHUNK, D), jnp.float32),
        mesh=mesh,
        scratch_types=[
            pltpu.VMEM((chunks_per_w, CHUNK), jnp.int32),
            pltpu.VMEM((chunks_per_w, CHUNK), jnp.int32),
            pltpu.VMEM((NBUF, CHUNK, 2 * D), jnp.float32),
            pltpu.VMEM((NBUF, CHUNK, D), jnp.float32),
            pltpu.VMEM((max_seq, D), jnp.float32),
            pltpu.SemaphoreType.DMA,
            pltpu.SemaphoreType.DMA,
        ],
        compiler_params=pltpu.CompilerParams(use_tc_tiling_on_sc=False),
    )
    def k(token_hbm, wt2_hbm, pos_hbm, out_hbm,
          idx_all, idx_sh, rows, bufs, pos_v, gsem, wsem):
        wid = lax.axis_index("s") * nc + lax.axis_index("c")
        chunk0 = wid * chunks_per_w
        pltpu.sync_copy(pos_hbm, pos_v)
        pltpu.sync_copy(token_hbm.at[pl.ds(chunk0, chunks_per_w)], idx_all)

        def shift_body(r, carry):
            for jj in (0, 16, 32, 48, 64, 80, 84):
                sl = pl.ds(jj, LANES)
                idx_sh[r, sl] = lax.shift_right_logical(idx_all[r, sl], 1)
            return carry

        lax.fori_loop(0, chunks_per_w, shift_body, 0)

        def issue_gather(c, b):
            pltpu.async_copy(wt2_hbm.at[idx_sh.at[c]], rows.at[b], gsem)

        def drain_g(b):
            pltpu.make_async_copy(
                wt2_hbm.at[pl.ds(0, CHUNK)], rows.at[b], gsem
            ).wait()

        def drain(sem, b):
            pltpu.make_async_copy(
                out_hbm.at[0, pl.ds(0, CHUNK)], bufs.at[b], sem
            ).wait()

        for t in range(NBUF - 1):
            issue_gather(t, t)

        def blk_body(blk, carry):
            for b in range(NBUF):
                c = blk * NBUF + b
                drain_g(b)
                half = (b % 2) * CHUNK

                def row_body(r, c2, _b=b, _half=half, _c=c):
                    h = (idx_all[_c, pl.ds(r, 1)][0] & 1) * D
                    pr = _half + r
                    for j in range(D // LANES):
                        sl = pl.ds(j * LANES, LANES)
                        bufs[_b, r, sl] = (
                            rows[_b, r, pl.ds(h + j * LANES, LANES)] * 8.0
                            + pos_v[pr, sl]
                        )
                    return c2

                lax.fori_loop(0, CHUNK, row_body, 0)
                seq = wid * seq_per_w + (c // 2)
                pltpu.async_copy(
                    bufs.at[b], out_hbm.at[seq, pl.ds(half, CHUNK)], wsem
                )
                nxt = c + NBUF - 1
                nb = (b + NBUF - 1) % NBUF

                @pl.when(nxt < chunks_per_w)
                def _():
                    @pl.when(c >= 1)
                    def _():
                        drain(wsem, nb)

                    issue_gather(nxt, nb)

            return carry

        lax.fori_loop(0, n_blks, blk_body, 0)

        for t in range(NBUF):
            drain(wsem, t)

    return k


def kernel(token, word_table, pos_table):
    b, s = token.shape
    vocab, d = word_table.shape
    max_seq = pos_table.shape[0]
    assert d == D and s % (2 * CHUNK) == 0 and max_seq == 2 * CHUNK
    assert vocab % 2 == 0
    token2 = token.reshape(-1, CHUNK)
    wt2 = word_table.reshape(vocab // 2, 2 * D)
    return _build(token2.shape[0], vocab, max_seq)(token2, wt2, pos_table)


# speedup vs baseline: 1.2556x; 1.2556x over previous
import functools

import jax
import jax.numpy as jnp
from jax import lax
from jax.experimental import pallas as pl
from jax.experimental.pallas import tpu as pltpu
from jax.experimental.pallas import tpu_sc as plsc

D = 64
LANES = 16
CHUNK = 100
NBUF = 8


@functools.lru_cache(maxsize=None)
def _build(n_chunks: int, vocab: int, max_seq: int):
    mesh = plsc.VectorSubcoreMesh(core_axis_name="c", subcore_axis_name="s")
    info = plsc.get_sparse_core_info()
    nc, ns = info.num_cores, info.num_subcores
    nw = nc * ns
    assert n_chunks % (nw * NBUF) == 0
    chunks_per_w = n_chunks // nw
    n_blks = chunks_per_w // NBUF
    n_seq = n_chunks // 2
    seq_per_w = n_seq // nw

    @functools.partial(
        pl.kernel,
        out_type=jax.ShapeDtypeStruct((n_seq, 2 * CHUNK, D), jnp.float32),
        mesh=mesh,
        scratch_types=[
            pltpu.VMEM((chunks_per_w, CHUNK), jnp.int32),
            pltpu.VMEM((NBUF, CHUNK, D), jnp.float32),
            pltpu.VMEM((max_seq, D), jnp.float32),
            pltpu.SemaphoreType.DMA,
            pltpu.SemaphoreType.DMA,
        ],
        compiler_params=pltpu.CompilerParams(use_tc_tiling_on_sc=False),
    )
    def k(token_hbm, wt_hbm, pos_hbm, out_hbm, idx_all, bufs, pos_v, gsem, wsem):
        wid = lax.axis_index("s") * nc + lax.axis_index("c")
        chunk0 = wid * chunks_per_w
        pltpu.sync_copy(pos_hbm, pos_v)
        pltpu.sync_copy(token_hbm.at[pl.ds(chunk0, chunks_per_w)], idx_all)

        def issue_gather(c, b):
            pltpu.async_copy(wt_hbm.at[idx_all.at[c]], bufs.at[b], gsem)

        def drain(sem, b):
            pltpu.make_async_copy(
                out_hbm.at[0, pl.ds(0, CHUNK)], bufs.at[b], sem
            ).wait()

        for t in range(NBUF - 1):
            issue_gather(t, t)

        def blk_body(blk, carry):
            for b in range(NBUF):
                c = blk * NBUF + b
                drain(gsem, b)
                half = (b % 2) * CHUNK

                def row_body(rr, c2, _b=b, _half=half):
                    for u in range(2):
                        r = rr * 2 + u
                        pr = _half + r
                        for j in range(D // LANES):
                            sl = pl.ds(j * LANES, LANES)
                            bufs[_b, r, sl] = (
                                bufs[_b, r, sl] * 8.0 + pos_v[pr, sl]
                            )
                    return c2

                lax.fori_loop(0, CHUNK // 2, row_body, 0)
                seq = wid * seq_per_w + (c // 2)
                pltpu.async_copy(
                    bufs.at[b], out_hbm.at[seq, pl.ds(half, CHUNK)], wsem
                )
                nxt = c + NBUF - 1
                nb = (b + NBUF - 1) % NBUF

                @pl.when(nxt < chunks_per_w)
                def _():
                    @pl.when(c >= 1)
                    def _():
                        drain(wsem, nb)

                    issue_gather(nxt, nb)

            return carry

        lax.fori_loop(0, n_blks, blk_body, 0)

        for t in range(NBUF):
            drain(wsem, t)

    return k


def kernel(token, word_table, pos_table):
    b, s = token.shape
    vocab, d = word_table.shape
    max_seq = pos_table.shape[0]
    assert d == D and s % (2 * CHUNK) == 0 and max_seq == 2 * CHUNK
    token2 = token.reshape(-1, CHUNK)
    return _build(token2.shape[0], vocab, max_seq)(token2, word_table, pos_table)


# speedup vs baseline: 1.2609x; 1.0042x over previous
import functools

import jax
import jax.numpy as jnp
from jax import lax
from jax.experimental import pallas as pl
from jax.experimental.pallas import tpu as pltpu
from jax.experimental.pallas import tpu_sc as plsc

D = 64
LANES = 16
SEQ = 200
NBUF = 4


@functools.lru_cache(maxsize=None)
def _build(n_seq: int, vocab: int, max_seq: int):
    mesh = plsc.VectorSubcoreMesh(core_axis_name="c", subcore_axis_name="s")
    info = plsc.get_sparse_core_info()
    nc, ns = info.num_cores, info.num_subcores
    nw = nc * ns
    assert n_seq % (nw * NBUF) == 0
    seq_per_w = n_seq // nw
    n_blks = seq_per_w // NBUF

    @functools.partial(
        pl.kernel,
        out_type=jax.ShapeDtypeStruct((n_seq, SEQ, D), jnp.float32),
        mesh=mesh,
        scratch_types=[
            pltpu.VMEM((seq_per_w, SEQ), jnp.int32),
            pltpu.VMEM((NBUF, SEQ, D), jnp.float32),
            pltpu.VMEM((max_seq, D), jnp.float32),
            pltpu.SemaphoreType.DMA,
            pltpu.SemaphoreType.DMA,
        ],
        compiler_params=pltpu.CompilerParams(use_tc_tiling_on_sc=False),
    )
    def k(token_hbm, wt_hbm, pos_hbm, out_hbm, idx_all, bufs, pos_v, gsem, wsem):
        wid = lax.axis_index("s") * nc + lax.axis_index("c")
        seq0 = wid * seq_per_w
        pltpu.sync_copy(pos_hbm, pos_v)
        pltpu.sync_copy(token_hbm.at[pl.ds(seq0, seq_per_w)], idx_all)

        def issue_gather(c, b):
            pltpu.async_copy(wt_hbm.at[idx_all.at[c]], bufs.at[b], gsem)

        def drain(sem, b):
            pltpu.make_async_copy(out_hbm.at[0], bufs.at[b], sem).wait()

        for t in range(NBUF - 1):
            issue_gather(t, t)

        def blk_body(blk, carry):
            for b in range(NBUF):
                c = blk * NBUF + b
                drain(gsem, b)

                def row_body(rr, c2, _b=b):
                    for u in range(2):
                        r = rr * 2 + u
                        for j in range(D // LANES):
                            sl = pl.ds(j * LANES, LANES)
                            bufs[_b, r, sl] = (
                                bufs[_b, r, sl] * 8.0 + pos_v[r, sl]
                            )
                    return c2

                lax.fori_loop(0, SEQ // 2, row_body, 0)
                pltpu.async_copy(bufs.at[b], out_hbm.at[seq0 + c], wsem)

                nxt = c + NBUF - 1
                nb = (b + NBUF - 1) % NBUF

                @pl.when(nxt < seq_per_w)
                def _():
                    @pl.when(c >= 1)
                    def _():
                        drain(wsem, nb)

                    issue_gather(nxt, nb)

            return carry

        lax.fori_loop(0, n_blks, blk_body, 0)

        for t in range(NBUF):
            drain(wsem, t)

    return k


def kernel(token, word_table, pos_table):
    b, s = token.shape
    vocab, d = word_table.shape
    max_seq = pos_table.shape[0]
    assert d == D and s == SEQ and max_seq == SEQ
    return _build(b, vocab, max_seq)(token, word_table, pos_table)
